# baseline (speedup 1.0000x reference)
import jax
import jax.numpy as jnp
from jax.experimental import pallas as pl
from jax.experimental.pallas import tpu as pltpu

_B = 256
_NO = 1000
_NM = 64
_F = 8
_EC = 64

_ER = _NO * _EC
_BRR = 1000
_EG = _ER // _BRR
_KEEP = 50
_ECH = 200

_OB = 250
_OG = _NO // _OB
_OCH = 50

_N_EDGE = float(_B * _NO * _EC)


def _edge_kernel(edge_ref, out_ref, keep_ref, stats_ref):
    p = pl.program_id(0)
    i = pl.program_id(1)

    @pl.when((p == 0) & (i == 0))
    def _init():
        stats_ref[0] = 0.0
        stats_ref[1] = 0.0

    @pl.when(p == 0)
    def _accumulate():
        s = jnp.float32(0.0)
        q = jnp.float32(0.0)
        for k in range(_BRR // _ECH):
            c = edge_ref[k * _ECH:(k + 1) * _ECH, :]
            s = s + jnp.sum(c)
            q = q + jnp.sum(c * c)
        stats_ref[0] += s
        stats_ref[1] += q

        @pl.when(i < _KEEP)
        def _park():
            for k in range(_BRR // _ECH):
                keep_ref[pl.ds(i * _BRR + k * _ECH, _ECH), :] = (
                    edge_ref[k * _ECH:(k + 1) * _ECH, :])

    @pl.when(p == 1)
    def _normalize():
        s = stats_ref[0]
        q = stats_ref[1]
        gm = s / _N_EDGE
        var = (q - _N_EDGE * gm * gm) * (1.0 / (_N_EDGE - 1.0))
        a = 1.0 / (jnp.sqrt(var) + 1e-5)
        b = -gm * a

        @pl.when(i < _KEEP)
        def _from_vmem():
            for k in range(_BRR // _ECH):
                out_ref[k * _ECH:(k + 1) * _ECH, :] = (
                    keep_ref[pl.ds(i * _BRR + k * _ECH, _ECH), :] * a + b)

        @pl.when(i >= _KEEP)
        def _from_hbm():
            for k in range(_BRR // _ECH):
                sl = slice(k * _ECH, (k + 1) * _ECH)
                out_ref[sl, :] = edge_ref[sl, :] * a + b


def _feat_kernel(opes_ref, mas_ref, opes_out_ref, mas_out_ref,
                 keep_ref, sum_ref, sq_ref):
    p = pl.program_id(0)
    i = pl.program_id(1)

    @pl.when((p == 0) & (i == 0))
    def _mas():
        y = mas_ref[...]
        m = jnp.mean(y, axis=0, keepdims=True)
        c = y - m
        v = jnp.sum(c * c, axis=0, keepdims=True) * (1.0 / (_NM - 1))
        mas_out_ref[...] = c / (jnp.sqrt(v) + 1e-5)

    @pl.when(p == 0)
    def _accumulate():
        @pl.when(i == 0)
        def _init():
            sum_ref[...] = jnp.zeros((_F, _B), jnp.float32)
            sq_ref[...] = jnp.zeros((_F, _B), jnp.float32)

        for k in range(_OB // _OCH):
            c = opes_ref[k * _OCH:(k + 1) * _OCH]
            sum_ref[...] += jnp.sum(c, axis=0)
            sq_ref[...] += jnp.sum(c * c, axis=0)
            keep_ref[pl.ds(i * _OB + k * _OCH, _OCH)] = c

    @pl.when(p == 1)
    def _normalize():
        m = sum_ref[...] * (1.0 / _NO)
        var = (sq_ref[...] - _NO * m * m) * (1.0 / (_NO - 1))
        inv = 1.0 / (jnp.sqrt(var) + 1e-5)
        for k in range(_OB // _OCH):
            c = keep_ref[pl.ds(i * _OB + k * _OCH, _OCH)]
            opes_out_ref[k * _OCH:(k + 1) * _OCH] = (c - m) * inv


@jax.jit
def kernel(batch_opr_features, batch_station_features, batch_edge_features):
    edge_t = jnp.transpose(batch_edge_features, (1, 2, 0)).reshape(_ER, _B)
    opes_t = jnp.transpose(batch_opr_features, (1, 2, 0))
    mas_t = jnp.transpose(batch_station_features, (1, 2, 0))

    edge_out = pl.pallas_call(
        _edge_kernel,
        grid=(2, _EG),
        in_specs=[
            pl.BlockSpec(
                (_BRR, _B),
                lambda p, i: (jnp.where((p == 0) | (i >= _KEEP), i, 0), 0)),
        ],
        out_specs=pl.BlockSpec(
            (_BRR, _B), lambda p, i: (jnp.where(p == 1, i, 0), 0)),
        out_shape=jax.ShapeDtypeStruct((_ER, _B), jnp.float32),
        scratch_shapes=[
            pltpu.VMEM((_KEEP * _BRR, _B), jnp.float32),
            pltpu.SMEM((2,), jnp.float32),
        ],
        compiler_params=pltpu.CompilerParams(
            dimension_semantics=("arbitrary", "arbitrary"),
        ),
    )(edge_t)

    opes_out, mas_out = pl.pallas_call(
        _feat_kernel,
        grid=(2, _OG),
        in_specs=[
            pl.BlockSpec(
                (_OB, _F, _B),
                lambda p, i: (jnp.where(p == 0, i, 0), 0, 0)),
            pl.BlockSpec((_NM, _F, _B), lambda p, i: (0, 0, 0)),
        ],
        out_specs=[
            pl.BlockSpec(
                (_OB, _F, _B),
                lambda p, i: (jnp.where(p == 1, i, 0), 0, 0)),
            pl.BlockSpec((_NM, _F, _B), lambda p, i: (0, 0, 0)),
        ],
        out_shape=[
            jax.ShapeDtypeStruct((_NO, _F, _B), jnp.float32),
            jax.ShapeDtypeStruct((_NM, _F, _B), jnp.float32),
        ],
        scratch_shapes=[
            pltpu.VMEM((_NO, _F, _B), jnp.float32),
            pltpu.VMEM((_F, _B), jnp.float32),
            pltpu.VMEM((_F, _B), jnp.float32),
        ],
        compiler_params=pltpu.CompilerParams(
            dimension_semantics=("arbitrary", "arbitrary"),
        ),
    )(opes_t, mas_t)

    return (
        jnp.transpose(opes_out, (2, 0, 1)),
        jnp.transpose(mas_out, (2, 0, 1)),
        jnp.transpose(edge_out.reshape(_NO, _EC, _B), (2, 0, 1)),
    )


# speedup vs baseline: 1.5318x; 1.5318x over previous
import jax
import jax.numpy as jnp
from jax.experimental import pallas as pl
from jax.experimental.pallas import tpu as pltpu

_B = 256
_NO = 1000
_NM = 64
_F = 8
_EC = 64

_ER = _NO * _EC
_BR = 8000
_GRID = _ER // _BR
_ECH = 400
_OCH = 100

_N_EDGE = float(_B * _NO * _EC)


def _normalize_resident(x_ref, out_ref, count, chunk):
    n = count // chunk
    m = jnp.zeros((1, _F, _B), jnp.float32)
    for k in range(n):
        m = m + jnp.sum(x_ref[k * chunk:(k + 1) * chunk], axis=0,
                        keepdims=True)
    m = m * (1.0 / count)
    q = jnp.zeros((1, _F, _B), jnp.float32)
    for k in range(n):
        c = x_ref[k * chunk:(k + 1) * chunk] - m
        q = q + jnp.sum(c * c, axis=0, keepdims=True)
    inv = 1.0 / (jnp.sqrt(q * (1.0 / (count - 1))) + 1e-5)
    for k in range(n):
        sl = slice(k * chunk, (k + 1) * chunk)
        out_ref[sl] = (x_ref[sl] - m) * inv


def _stats_opes_kernel(edge_ref, opes_ref, mas_ref,
                       opes_out_ref, mas_out_ref, stats_ref):
    step = pl.program_id(0)

    @pl.when(step == 0)
    def _init():
        stats_ref[0] = 0.0
        stats_ref[1] = 0.0

    s = jnp.float32(0.0)
    q = jnp.float32(0.0)
    for k in range(_BR // _ECH):
        c = edge_ref[k * _ECH:(k + 1) * _ECH, :]
        s = s + jnp.sum(c)
        q = q + jnp.sum(c * c)
    stats_ref[0] += s
    stats_ref[1] += q

    @pl.when(step == 0)
    def _features():
        _normalize_resident(opes_ref, opes_out_ref, _NO, _OCH)
        _normalize_resident(mas_ref, mas_out_ref, _NM, _NM)


def _edge_norm_kernel(stats_ref, edge_ref, edge_out_ref):
    s = stats_ref[0]
    q = stats_ref[1]
    gm = s / _N_EDGE
    var = (q - _N_EDGE * gm * gm) * (1.0 / (_N_EDGE - 1.0))
    a = 1.0 / (jnp.sqrt(var) + 1e-5)
    b = -gm * a
    for k in range(_BR // _ECH):
        sl = slice(k * _ECH, (k + 1) * _ECH)
        edge_out_ref[sl, :] = edge_ref[sl, :] * a + b


@jax.jit
def kernel(batch_opr_features, batch_station_features, batch_edge_features):
    edge_t = jnp.transpose(batch_edge_features, (1, 2, 0)).reshape(_ER, _B)
    opes_t = jnp.transpose(batch_opr_features, (1, 2, 0))
    mas_t = jnp.transpose(batch_station_features, (1, 2, 0))

    opes_out, mas_out, stats = pl.pallas_call(
        _stats_opes_kernel,
        grid=(_GRID,),
        in_specs=[
            pl.BlockSpec((_BR, _B), lambda i: (i, 0)),
            pl.BlockSpec((_NO, _F, _B), lambda i: (0, 0, 0)),
            pl.BlockSpec((_NM, _F, _B), lambda i: (0, 0, 0)),
        ],
        out_specs=[
            pl.BlockSpec((_NO, _F, _B), lambda i: (0, 0, 0)),
            pl.BlockSpec((_NM, _F, _B), lambda i: (0, 0, 0)),
            pl.BlockSpec(memory_space=pltpu.SMEM),
        ],
        out_shape=[
            jax.ShapeDtypeStruct((_NO, _F, _B), jnp.float32),
            jax.ShapeDtypeStruct((_NM, _F, _B), jnp.float32),
            jax.ShapeDtypeStruct((2,), jnp.float32),
        ],
        compiler_params=pltpu.CompilerParams(
            dimension_semantics=("arbitrary",),
        ),
    )(edge_t, opes_t, mas_t)

    edge_out = pl.pallas_call(
        _edge_norm_kernel,
        grid=(_GRID,),
        in_specs=[
            pl.BlockSpec(memory_space=pltpu.SMEM),
            pl.BlockSpec((_BR, _B), lambda i: (i, 0)),
        ],
        out_specs=pl.BlockSpec((_BR, _B), lambda i: (i, 0)),
        out_shape=jax.ShapeDtypeStruct((_ER, _B), jnp.float32),
        compiler_params=pltpu.CompilerParams(
            dimension_semantics=("arbitrary",),
        ),
    )(stats, edge_t)

    return (
        jnp.transpose(opes_out, (2, 0, 1)),
        jnp.transpose(mas_out, (2, 0, 1)),
        jnp.transpose(edge_out.reshape(_NO, _EC, _B), (2, 0, 1)),
    )


# speedup vs baseline: 1.5607x; 1.0188x over previous
import jax
import jax.numpy as jnp
from jax.experimental import pallas as pl
from jax.experimental.pallas import tpu as pltpu

_B = 256
_NO = 1000
_NM = 64
_F = 8
_EC = 64

_ER = _NO * _EC
_BR = 12800
_GRID = _ER // _BR
_ECH = 400
_OCH = 100

_N_EDGE = float(_B * _NO * _EC)


def _normalize_resident(x_ref, out_ref, count, chunk):
    n = count // chunk
    m = jnp.zeros((1, _F, _B), jnp.float32)
    for k in range(n):
        m = m + jnp.sum(x_ref[k * chunk:(k + 1) * chunk], axis=0,
                        keepdims=True)
    m = m * (1.0 / count)
    q = jnp.zeros((1, _F, _B), jnp.float32)
    for k in range(n):
        c = x_ref[k * chunk:(k + 1) * chunk] - m
        q = q + jnp.sum(c * c, axis=0, keepdims=True)
    inv = 1.0 / (jnp.sqrt(q * (1.0 / (count - 1))) + 1e-5)
    for k in range(n):
        sl = slice(k * chunk, (k + 1) * chunk)
        out_ref[sl] = (x_ref[sl] - m) * inv


def _stats_opes_kernel(edge_ref, opes_ref, mas_ref,
                       opes_out_ref, mas_out_ref, stats_ref):
    step = pl.program_id(0)

    @pl.when(step == 0)
    def _init():
        stats_ref[0] = 0.0
        stats_ref[1] = 0.0

    s = jnp.float32(0.0)
    q = jnp.float32(0.0)
    for k in range(_BR // _ECH):
        c = edge_ref[k * _ECH:(k + 1) * _ECH, :]
        s = s + jnp.sum(c)
        q = q + jnp.sum(c * c)
    stats_ref[0] += s
    stats_ref[1] += q

    @pl.when(step == 0)
    def _features():
        _normalize_resident(opes_ref, opes_out_ref, _NO, _OCH)
        _normalize_resident(mas_ref, mas_out_ref, _NM, _NM)


def _edge_norm_kernel(stats_ref, edge_ref, edge_out_ref):
    s = stats_ref[0]
    q = stats_ref[1]
    gm = s / _N_EDGE
    var = (q - _N_EDGE * gm * gm) * (1.0 / (_N_EDGE - 1.0))
    a = 1.0 / (jnp.sqrt(var) + 1e-5)
    b = -gm * a
    for k in range(_BR // _ECH):
        sl = slice(k * _ECH, (k + 1) * _ECH)
        edge_out_ref[sl, :] = edge_ref[sl, :] * a + b


@jax.jit
def kernel(batch_opr_features, batch_station_features, batch_edge_features):
    edge_t = jnp.transpose(batch_edge_features, (1, 2, 0)).reshape(_ER, _B)
    opes_t = jnp.transpose(batch_opr_features, (1, 2, 0))
    mas_t = jnp.transpose(batch_station_features, (1, 2, 0))

    opes_out, mas_out, stats = pl.pallas_call(
        _stats_opes_kernel,
        grid=(_GRID,),
        in_specs=[
            pl.BlockSpec((_BR, _B), lambda i: (i, 0)),
            pl.BlockSpec((_NO, _F, _B), lambda i: (0, 0, 0)),
            pl.BlockSpec((_NM, _F, _B), lambda i: (0, 0, 0)),
        ],
        out_specs=[
            pl.BlockSpec((_NO, _F, _B), lambda i: (0, 0, 0)),
            pl.BlockSpec((_NM, _F, _B), lambda i: (0, 0, 0)),
            pl.BlockSpec(memory_space=pltpu.SMEM),
        ],
        out_shape=[
            jax.ShapeDtypeStruct((_NO, _F, _B), jnp.float32),
            jax.ShapeDtypeStruct((_NM, _F, _B), jnp.float32),
            jax.ShapeDtypeStruct((2,), jnp.float32),
        ],
        compiler_params=pltpu.CompilerParams(
            dimension_semantics=("arbitrary",),
        ),
    )(edge_t, opes_t, mas_t)

    edge_out = pl.pallas_call(
        _edge_norm_kernel,
        grid=(_GRID,),
        in_specs=[
            pl.BlockSpec(memory_space=pltpu.SMEM),
            pl.BlockSpec((_BR, _B), lambda i: (i, 0)),
        ],
        out_specs=pl.BlockSpec((_BR, _B), lambda i: (i, 0)),
        out_shape=jax.ShapeDtypeStruct((_ER, _B), jnp.float32),
        compiler_params=pltpu.CompilerParams(
            dimension_semantics=("arbitrary",),
        ),
    )(stats, edge_t)

    return (
        jnp.transpose(opes_out, (2, 0, 1)),
        jnp.transpose(mas_out, (2, 0, 1)),
        jnp.transpose(edge_out.reshape(_NO, _EC, _B), (2, 0, 1)),
    )
